# baseline (speedup 1.0000x reference)
import functools

import jax
import jax.numpy as jnp
from jax import lax
from jax.experimental import pallas as pl
from jax.experimental.pallas import tpu as pltpu
from jax.experimental.pallas import tpu_sc as plsc

N0, N1, N2 = 50000, 10000, 2000
E1, E2 = 160000, 32000
D = 128
H = 2
C = 64
ED = 23

CH = 128
ACCW = 80
ROWB = 128
MMB = 400


def _kv_body(x_ref, w_ref, b_ref, o_ref):
    x = x_ref[...].astype(jnp.bfloat16)
    o_ref[0] = jnp.dot(x, w_ref[0], preferred_element_type=jnp.float32) + b_ref[0]
    o_ref[1] = jnp.dot(x, w_ref[1], preferred_element_type=jnp.float32) + b_ref[1]


def _proj_kv(x, Wk, bk, Wv, bv):
    n = x.shape[0]
    wh = jnp.stack([
        jnp.concatenate([Wk[:, :C], Wv[:, :C]], axis=1),
        jnp.concatenate([Wk[:, C:], Wv[:, C:]], axis=1),
    ])
    bh = jnp.stack([
        jnp.concatenate([bk[:C], bv[:C]]),
        jnp.concatenate([bk[C:], bv[C:]]),
    ]).reshape(2, 1, 2 * C)
    wh = wh.astype(jnp.bfloat16)
    return pl.pallas_call(
        _kv_body,
        grid=(n // MMB,),
        in_specs=[
            pl.BlockSpec((MMB, D), lambda i: (i, 0)),
            pl.BlockSpec((2, D, 2 * C), lambda i: (0, 0, 0)),
            pl.BlockSpec((2, 1, 2 * C), lambda i: (0, 0, 0)),
        ],
        out_specs=pl.BlockSpec((2, MMB, 2 * C), lambda i: (0, i, 0)),
        out_shape=jax.ShapeDtypeStruct((2, n, 2 * C), jnp.float32),
    )(x, wh, bh)


def _qxr_body(x_ref, w_ref, b_ref, q_ref, xr_ref):
    y = (jnp.dot(x_ref[...].astype(jnp.bfloat16), w_ref[...],
                 preferred_element_type=jnp.float32) + b_ref[...])
    q_ref[0] = y[:, :C]
    q_ref[1] = y[:, C:2 * C]
    xr_ref[...] = y[:, 2 * C:]


def _proj_qxr(x, Wq, bq, Ws, bs):
    n = x.shape[0]
    dxr = Ws.shape[1]
    wc = jnp.concatenate([Wq[:, :C], Wq[:, C:], Ws], axis=1).astype(jnp.bfloat16)
    bc = jnp.concatenate([bq[:C], bq[C:], bs]).reshape(1, -1)
    dout = 2 * C + dxr
    return pl.pallas_call(
        _qxr_body,
        grid=(n // MMB,),
        in_specs=[
            pl.BlockSpec((MMB, D), lambda i: (i, 0)),
            pl.BlockSpec((D, dout), lambda i: (0, 0)),
            pl.BlockSpec((1, dout), lambda i: (0, 0)),
        ],
        out_specs=[
            pl.BlockSpec((2, MMB, C), lambda i: (0, i, 0)),
            pl.BlockSpec((MMB, dxr), lambda i: (i, 0)),
        ],
        out_shape=[
            jax.ShapeDtypeStruct((2, n, C), jnp.float32),
            jax.ShapeDtypeStruct((n, dxr), jnp.float32),
        ],
    )(x, wc, bc)


def _sc_edge_body(nchunks, nblocks, n_src, n_q, nj,
                  kv_h, q_h, we_h, idx3_h, out_h,
                  acc_s, wevm,
                  kvb0, kvb1, qrb0, qrb1, srow0, srow1,
                  idxb0, idxb1, sidx0, sidx1, ipa, ipb,
                  sem_kv0, sem_kv1, sem_q0, sem_q1, sem_s0, sem_s1,
                  sem_ia, sem_ib):
    c = lax.axis_index("c")
    s = lax.axis_index("s")
    zero = jnp.zeros((16,), jnp.float32)
    lane = lax.iota(jnp.int32, 16)
    c_src = c * n_src
    c_dst = c * n_q
    c_eid = c * ED
    base = s * nj
    B0 = (kvb0, qrb0, srow0, idxb0, sidx0, sem_kv0, sem_q0, sem_s0)
    B1 = (kvb1, qrb1, srow1, idxb1, sidx1, sem_kv1, sem_q1, sem_s1)
    IP = ((ipa, sem_ia), (ipb, sem_ib))

    pltpu.sync_copy(we_h, wevm)

    def _zero_row(r, _):
        for j in range(ACCW // 16):
            srow0[r, pl.ds(16 * j, 16)] = zero
        return 0
    lax.fori_loop(0, ROWB, _zero_row, 0)

    def _zero_blk(w, _):
        b = w * 16 + s

        @pl.when(b < nblocks)
        def _():
            pltpu.sync_copy(srow0, acc_s.at[pl.ds(b * ROWB, ROWB)])
        return 0
    lax.fori_loop(0, -(-nblocks // 16), _zero_blk, 0)
    plsc.subcore_barrier()

    def _issue_pair(q2, ip, sem):
        cid = base + 2 * q2

        @pl.when((2 * q2 < nj) & (cid < nchunks))
        def _():
            pltpu.make_async_copy(idx3_h.at[pl.ds(cid, 2)], ip, sem).start()

    def _stage(j, pr, ipsel, B):
        kvb, qrb, srow, idxb, sidx, sem_kv, sem_q, sem_s = B
        ip, sem_i = IP[ipsel]
        cid = base + j

        @pl.when((j < nj) & (cid < nchunks))
        def _():
            if pr == 0:
                pltpu.make_async_copy(
                    idx3_h.at[pl.ds(cid, 2)], ip, sem_i).wait()
            for r in range(CH // 16):
                sl = pl.ds(16 * r, 16)
                dstr = ip[pr, 1, sl]
                idxb[0, sl] = ip[pr, 0, sl] + c_src
                idxb[1, sl] = dstr
                idxb[2, sl] = dstr + c_dst
                idxb[3, sl] = ip[pr, 2, sl] + c_eid
            pltpu.make_async_copy(kv_h.at[idxb.at[0]], kvb, sem_kv).start()
            pltpu.make_async_copy(q_h.at[idxb.at[2]], qrb, sem_q).start()
        if pr == 1:
            _issue_pair((j - 1) // 2 + 2, ip, sem_i)

    def _wait_gathers(j, B):
        kvb, qrb, srow, idxb, sidx, sem_kv, sem_q, sem_s = B
        cid = base + j

        @pl.when((j < nj) & (cid < nchunks))
        def _():
            pltpu.make_async_copy(kv_h.at[idxb.at[0]], kvb, sem_kv).wait()
            pltpu.make_async_copy(q_h.at[idxb.at[2]], qrb, sem_q).wait()

    def _wait_scatter(j, B):
        kvb, qrb, srow, idxb, sidx, sem_kv, sem_q, sem_s = B
        cid = base + j

        @pl.when((j >= 0) & (j < nj) & (cid < nchunks))
        def _():
            pltpu.make_async_copy(srow, acc_s.at[sidx], sem_s).wait()

    def _compute_scatter(j, B):
        kvb, qrb, srow, idxb, sidx, sem_kv, sem_q, sem_s = B
        cid = base + j

        @pl.when((j < nj) & (cid < nchunks))
        def _():
            @plsc.parallel_loop(0, CH, unroll=16)
            def _edge(ei):
                eid16 = plsc.load_gather(
                    idxb.at[3], [jnp.full((16,), ei, jnp.int32)])
                er = [plsc.load_gather(wevm, [eid16, lane + 16 * jj])
                      for jj in range(4)]
                acc = zero
                for jj in range(4):
                    sl = pl.ds(16 * jj, 16)
                    acc = acc + (kvb[ei, sl] + er[jj]) * qrb[ei, sl]
                a = jnp.sum(acc) * 0.125
                ex = jnp.exp(jnp.full((16,), a, jnp.float32))
                srow[ei, pl.ds(64, 16)] = jnp.where(lane == 0, ex, zero)
                for jj in range(4):
                    sl = pl.ds(16 * jj, 16)
                    slv = pl.ds(64 + 16 * jj, 16)
                    srow[ei, sl] = (kvb[ei, slv] + er[jj]) * ex

            for r in range(CH // 16):
                sl = pl.ds(16 * r, 16)
                sidx[sl] = idxb[1, sl]
            pltpu.make_async_copy(srow, acc_s.at[sidx], sem_s).start(add=True)

    T = -(-nj // 4)
    _issue_pair(0, ipa, sem_ia)
    _issue_pair(1, ipb, sem_ib)
    _stage(0, 0, 0, B0)

    def _pipe(t, _):
        j0 = 4 * t
        for k in range(4):
            j = j0 + k
            bufs = (B0, B1)[k % 2]
            nbufs = (B1, B0)[k % 2]
            _stage(j + 1, (k + 1) % 2, ((k + 1) // 2) % 2, nbufs)
            _wait_gathers(j, bufs)
            _wait_scatter(j - 2, bufs)
            _compute_scatter(j, bufs)
        return 0

    lax.fori_loop(0, T, _pipe, 0)
    _wait_scatter(4 * T - 2, B0)
    _wait_scatter(4 * T - 1, B1)
    plsc.subcore_barrier()

    def _wr_blk(w, _):
        b = w * 16 + s

        @pl.when(b < nblocks)
        def _():
            rs = pl.ds(b * ROWB, ROWB)
            pltpu.sync_copy(acc_s.at[rs], out_h.at[c].at[rs])
        return 0
    lax.fori_loop(0, -(-nblocks // 16), _wr_blk, 0)


def _sc_edge(kvh, qh, weh, idx3, n_dst):
    nchunks = idx3.shape[0] - 1
    n_src = kvh.shape[0] // 2
    n_q = qh.shape[0] // 2
    nblocks = -(-n_dst // ROWB)
    nj = -(-nchunks // 16)
    assert n_q == n_dst

    mesh = plsc.VectorSubcoreMesh(core_axis_name="c", subcore_axis_name="s",
                                  num_cores=2, num_subcores=16)
    body = functools.partial(_sc_edge_body, nchunks, nblocks, n_src, n_q, nj)
    kfn = pl.kernel(
        body,
        out_type=jax.ShapeDtypeStruct((2, nblocks * ROWB, ACCW), jnp.float32),
        mesh=mesh,
        compiler_params=pltpu.CompilerParams(use_tc_tiling_on_sc=False,
                                             needs_layout_passes=False),
        scratch_types=[
            pltpu.VMEM_SHARED((nblocks * ROWB, ACCW), jnp.float32),
            pltpu.VMEM((2 * ED, C), jnp.float32),
            pltpu.VMEM((CH, 2 * C), jnp.float32),
            pltpu.VMEM((CH, 2 * C), jnp.float32),
            pltpu.VMEM((CH, C), jnp.float32),
            pltpu.VMEM((CH, C), jnp.float32),
            pltpu.VMEM((CH, ACCW), jnp.float32),
            pltpu.VMEM((CH, ACCW), jnp.float32),
            pltpu.VMEM((4, CH), jnp.int32),
            pltpu.VMEM((4, CH), jnp.int32),
            pltpu.VMEM((CH,), jnp.int32),
            pltpu.VMEM((CH,), jnp.int32),
            pltpu.VMEM((2, 3, CH), jnp.int32),
            pltpu.VMEM((2, 3, CH), jnp.int32),
            pltpu.SemaphoreType.DMA,
            pltpu.SemaphoreType.DMA,
            pltpu.SemaphoreType.DMA,
            pltpu.SemaphoreType.DMA,
            pltpu.SemaphoreType.DMA,
            pltpu.SemaphoreType.DMA,
            pltpu.SemaphoreType.DMA,
            pltpu.SemaphoreType.DMA,
        ],
    )
    return kfn(kvh, qh, weh, idx3)


def _gate1_body(acc_ref, xr_ref, ga_ref, gb_ref, lg_ref, lb_ref, o_ref):
    a0 = acc_ref[0]
    a1 = acc_ref[1]
    o0 = a0[:, :64] / (a0[:, 64:65] + 1e-16)
    o1 = a1[:, :64] / (a1[:, 64:65] + 1e-16)
    out = jnp.concatenate([o0, o1], axis=1)
    xr = xr_ref[...]
    beta = jax.nn.sigmoid(
        jnp.sum(xr * ga_ref[...] + out * gb_ref[...], axis=1, keepdims=True))
    h = beta * xr + (1.0 - beta) * out
    mu = jnp.mean(h, axis=1, keepdims=True)
    var = jnp.mean(jnp.square(h - mu), axis=1, keepdims=True)
    h = (h - mu) * jax.lax.rsqrt(var + 1e-5) * lg_ref[...] + lb_ref[...]
    o_ref[...] = jnp.maximum(h, 0.0)


def _gate1(acc, xr, ga, gb, lg, lb, block_rows=1000):
    n = xr.shape[0]
    vec = lambda v: v.reshape(1, -1)
    return pl.pallas_call(
        _gate1_body,
        grid=(n // block_rows,),
        in_specs=[
            pl.BlockSpec((2, block_rows, ACCW), lambda i: (0, i, 0)),
            pl.BlockSpec((block_rows, D), lambda i: (i, 0)),
            pl.BlockSpec((1, D), lambda i: (0, 0)),
            pl.BlockSpec((1, D), lambda i: (0, 0)),
            pl.BlockSpec((1, D), lambda i: (0, 0)),
            pl.BlockSpec((1, D), lambda i: (0, 0)),
        ],
        out_specs=pl.BlockSpec((block_rows, D), lambda i: (i, 0)),
        out_shape=jax.ShapeDtypeStruct((n, D), jnp.float32),
    )(acc, xr, vec(ga), vec(gb), vec(lg), vec(lb))


def _gate2_body(acc_ref, xr_ref, ga_ref, gb_ref, o_ref):
    a0 = acc_ref[0]
    a1 = acc_ref[1]
    o0 = a0[:, :64] / (a0[:, 64:65] + 1e-16)
    o1 = a1[:, :64] / (a1[:, 64:65] + 1e-16)
    out = 0.5 * (o0 + o1)
    xr = xr_ref[...]
    beta = jax.nn.sigmoid(
        jnp.sum(xr * ga_ref[...] + out * gb_ref[...], axis=1, keepdims=True))
    o = beta * xr + (1.0 - beta) * out
    m = jnp.max(o, axis=1, keepdims=True)
    lse = m + jnp.log(jnp.sum(jnp.exp(o - m), axis=1, keepdims=True))
    o_ref[...] = o - lse


def _gate2(acc, xr, ga, gb, block_rows=1000):
    n = xr.shape[0]
    vec = lambda v: v.reshape(1, -1)
    return pl.pallas_call(
        _gate2_body,
        grid=(n // block_rows,),
        in_specs=[
            pl.BlockSpec((2, block_rows, ACCW), lambda i: (0, i, 0)),
            pl.BlockSpec((block_rows, C), lambda i: (i, 0)),
            pl.BlockSpec((1, C), lambda i: (0, 0)),
            pl.BlockSpec((1, C), lambda i: (0, 0)),
        ],
        out_specs=pl.BlockSpec((block_rows, C), lambda i: (i, 0)),
        out_shape=jax.ShapeDtypeStruct((n, C), jnp.float32),
    )(acc, xr, vec(ga), vec(gb))


def _edge_blocks(src, dst, eid):
    e = src.shape[0]
    blk = jnp.stack([src.reshape(e // CH, CH), dst.reshape(e // CH, CH),
                     eid.reshape(e // CH, CH)], axis=1)
    return jnp.pad(blk, ((0, 1), (0, 0), (0, 0)))


def _weh(We):
    return We.reshape(ED, 2, C).transpose(1, 0, 2).reshape(2 * ED, C)


def kernel(x, src1, dst1, eid1, src2, dst2, eid2,
           Wq1, bq1, Wk1, bk1, Wv1, bv1, We1, Ws1, bs1, Wb1, ln_g, ln_b,
           Wq2, bq2, Wk2, bk2, Wv2, bv2, We2, Ws2, bs2, Wb2):
    kvh1 = _proj_kv(x, Wk1, bk1, Wv1, bv1).reshape(2 * N0, 2 * C)
    qh1, xr1 = _proj_qxr(x[:N1], Wq1, bq1, Ws1, bs1)
    qh1 = qh1.reshape(2 * N1, C)
    acc1 = _sc_edge(kvh1, qh1, _weh(We1),
                    _edge_blocks(src1, dst1, eid1[:, 0]), N1)
    ga1 = Wb1[:D, 0] + Wb1[2 * D:, 0]
    gb1 = Wb1[D:2 * D, 0] - Wb1[2 * D:, 0]
    h = _gate1(acc1, xr1, ga1, gb1, ln_g, ln_b)

    kvh2 = _proj_kv(h, Wk2, bk2, Wv2, bv2).reshape(2 * N1, 2 * C)
    qh2, xr2 = _proj_qxr(h[:N2], Wq2, bq2, Ws2, bs2)
    qh2 = qh2.reshape(2 * N2, C)
    acc2 = _sc_edge(kvh2, qh2, _weh(We2),
                    _edge_blocks(src2, dst2, eid2[:, 0]), N2)
    ga2 = Wb2[:C, 0] + Wb2[2 * C:, 0]
    gb2 = Wb2[C:2 * C, 0] - Wb2[2 * C:, 0]
    return _gate2(acc2, xr2, ga2, gb2)


# speedup vs baseline: 1.0077x; 1.0077x over previous
import functools

import jax
import jax.numpy as jnp
from jax import lax
from jax.experimental import pallas as pl
from jax.experimental.pallas import tpu as pltpu
from jax.experimental.pallas import tpu_sc as plsc

N0, N1, N2 = 50000, 10000, 2000
E1, E2 = 160000, 32000
D = 128
H = 2
C = 64
ED = 23

CH = 128
ACCW = 80
ROWB = 128
MMB = 400


def _kv_body(x_ref, w_ref, b_ref, o_ref):
    x = x_ref[...]
    o_ref[0] = jnp.dot(x, w_ref[0], preferred_element_type=jnp.float32) + b_ref[0]
    o_ref[1] = jnp.dot(x, w_ref[1], preferred_element_type=jnp.float32) + b_ref[1]


def _proj_kv(x, Wk, bk, Wv, bv):
    n = x.shape[0]
    wh = jnp.stack([
        jnp.concatenate([Wk[:, :C], Wv[:, :C]], axis=1),
        jnp.concatenate([Wk[:, C:], Wv[:, C:]], axis=1),
    ])
    bh = jnp.stack([
        jnp.concatenate([bk[:C], bv[:C]]),
        jnp.concatenate([bk[C:], bv[C:]]),
    ]).reshape(2, 1, 2 * C)
    return pl.pallas_call(
        _kv_body,
        grid=(n // MMB,),
        in_specs=[
            pl.BlockSpec((MMB, D), lambda i: (i, 0)),
            pl.BlockSpec((2, D, 2 * C), lambda i: (0, 0, 0)),
            pl.BlockSpec((2, 1, 2 * C), lambda i: (0, 0, 0)),
        ],
        out_specs=pl.BlockSpec((2, MMB, 2 * C), lambda i: (0, i, 0)),
        out_shape=jax.ShapeDtypeStruct((2, n, 2 * C), jnp.float32),
    )(x, wh, bh)


def _qxr_body(x_ref, w_ref, b_ref, q_ref, xr_ref):
    y = (jnp.dot(x_ref[...], w_ref[...], preferred_element_type=jnp.float32)
         + b_ref[...])
    q_ref[0] = y[:, :C]
    q_ref[1] = y[:, C:2 * C]
    xr_ref[...] = y[:, 2 * C:]


def _proj_qxr(x, Wq, bq, Ws, bs):
    n = x.shape[0]
    dxr = Ws.shape[1]
    wc = jnp.concatenate([Wq[:, :C], Wq[:, C:], Ws], axis=1)
    bc = jnp.concatenate([bq[:C], bq[C:], bs]).reshape(1, -1)
    dout = 2 * C + dxr
    return pl.pallas_call(
        _qxr_body,
        grid=(n // MMB,),
        in_specs=[
            pl.BlockSpec((MMB, D), lambda i: (i, 0)),
            pl.BlockSpec((D, dout), lambda i: (0, 0)),
            pl.BlockSpec((1, dout), lambda i: (0, 0)),
        ],
        out_specs=[
            pl.BlockSpec((2, MMB, C), lambda i: (0, i, 0)),
            pl.BlockSpec((MMB, dxr), lambda i: (i, 0)),
        ],
        out_shape=[
            jax.ShapeDtypeStruct((2, n, C), jnp.float32),
            jax.ShapeDtypeStruct((n, dxr), jnp.float32),
        ],
    )(x, wc, bc)


def _sc_edge_body(nchunks, nblocks, n_src, n_q, nj,
                  kv_h, q_h, we_h, idx3_h, out_h,
                  acc_s, wevm,
                  kvb0, kvb1, qrb0, qrb1, srow0, srow1,
                  idxb0, idxb1, sidx0, sidx1, ipa, ipb,
                  sem_kv0, sem_kv1, sem_q0, sem_q1, sem_s0, sem_s1,
                  sem_ia, sem_ib):
    c = lax.axis_index("c")
    s = lax.axis_index("s")
    zero = jnp.zeros((16,), jnp.float32)
    lane = lax.iota(jnp.int32, 16)
    c_src = c * n_src
    c_dst = c * n_q
    c_eid = c * ED
    base = s * nj
    B0 = (kvb0, qrb0, srow0, idxb0, sidx0, sem_kv0, sem_q0, sem_s0)
    B1 = (kvb1, qrb1, srow1, idxb1, sidx1, sem_kv1, sem_q1, sem_s1)
    IP = ((ipa, sem_ia), (ipb, sem_ib))

    pltpu.sync_copy(we_h, wevm)

    def _zero_row(r, _):
        for j in range(ACCW // 16):
            srow0[r, pl.ds(16 * j, 16)] = zero
        return 0
    lax.fori_loop(0, ROWB, _zero_row, 0)

    def _zero_blk(w, _):
        b = w * 16 + s

        @pl.when(b < nblocks)
        def _():
            pltpu.sync_copy(srow0, acc_s.at[pl.ds(b * ROWB, ROWB)])
        return 0
    lax.fori_loop(0, -(-nblocks // 16), _zero_blk, 0)
    plsc.subcore_barrier()

    def _issue_pair(q2, ip, sem):
        cid = base + 2 * q2

        @pl.when((2 * q2 < nj) & (cid < nchunks))
        def _():
            pltpu.make_async_copy(idx3_h.at[pl.ds(cid, 2)], ip, sem).start()

    def _stage(j, pr, ipsel, B):
        kvb, qrb, srow, idxb, sidx, sem_kv, sem_q, sem_s = B
        ip, sem_i = IP[ipsel]
        cid = base + j

        @pl.when((j < nj) & (cid < nchunks))
        def _():
            if pr == 0:
                pltpu.make_async_copy(
                    idx3_h.at[pl.ds(cid, 2)], ip, sem_i).wait()
            for r in range(CH // 16):
                sl = pl.ds(16 * r, 16)
                dstr = ip[pr, 1, sl]
                idxb[0, sl] = ip[pr, 0, sl] + c_src
                idxb[1, sl] = dstr
                idxb[2, sl] = dstr + c_dst
                idxb[3, sl] = ip[pr, 2, sl] + c_eid
            pltpu.make_async_copy(kv_h.at[idxb.at[0]], kvb, sem_kv).start()
            pltpu.make_async_copy(q_h.at[idxb.at[2]], qrb, sem_q).start()
        if pr == 1:
            _issue_pair((j - 1) // 2 + 2, ip, sem_i)

    def _wait_gathers(j, B):
        kvb, qrb, srow, idxb, sidx, sem_kv, sem_q, sem_s = B
        cid = base + j

        @pl.when((j < nj) & (cid < nchunks))
        def _():
            pltpu.make_async_copy(kv_h.at[idxb.at[0]], kvb, sem_kv).wait()
            pltpu.make_async_copy(q_h.at[idxb.at[2]], qrb, sem_q).wait()

    def _wait_scatter(j, B):
        kvb, qrb, srow, idxb, sidx, sem_kv, sem_q, sem_s = B
        cid = base + j

        @pl.when((j >= 0) & (j < nj) & (cid < nchunks))
        def _():
            pltpu.make_async_copy(srow, acc_s.at[sidx], sem_s).wait()

    def _compute_scatter(j, B):
        kvb, qrb, srow, idxb, sidx, sem_kv, sem_q, sem_s = B
        cid = base + j

        @pl.when((j < nj) & (cid < nchunks))
        def _():
            @plsc.parallel_loop(0, CH, unroll=16)
            def _edge(ei):
                eid16 = plsc.load_gather(
                    idxb.at[3], [jnp.full((16,), ei, jnp.int32)])
                er = [plsc.load_gather(wevm, [eid16, lane + 16 * jj])
                      for jj in range(4)]
                acc = zero
                for jj in range(4):
                    sl = pl.ds(16 * jj, 16)
                    acc = acc + (kvb[ei, sl] + er[jj]) * qrb[ei, sl]
                a = jnp.sum(acc) * 0.125
                ex = jnp.exp(jnp.full((16,), a, jnp.float32))
                srow[ei, pl.ds(64, 16)] = jnp.where(lane == 0, ex, zero)
                for jj in range(4):
                    sl = pl.ds(16 * jj, 16)
                    slv = pl.ds(64 + 16 * jj, 16)
                    srow[ei, sl] = (kvb[ei, slv] + er[jj]) * ex

            for r in range(CH // 16):
                sl = pl.ds(16 * r, 16)
                sidx[sl] = idxb[1, sl]
            pltpu.make_async_copy(srow, acc_s.at[sidx], sem_s).start(add=True)

    T = -(-nj // 4)
    _issue_pair(0, ipa, sem_ia)
    _issue_pair(1, ipb, sem_ib)
    _stage(0, 0, 0, B0)

    def _pipe(t, _):
        j0 = 4 * t
        for k in range(4):
            j = j0 + k
            bufs = (B0, B1)[k % 2]
            nbufs = (B1, B0)[k % 2]
            _stage(j + 1, (k + 1) % 2, ((k + 1) // 2) % 2, nbufs)
            _wait_gathers(j, bufs)
            _wait_scatter(j - 2, bufs)
            _compute_scatter(j, bufs)
        return 0

    lax.fori_loop(0, T, _pipe, 0)
    _wait_scatter(4 * T - 2, B0)
    _wait_scatter(4 * T - 1, B1)
    plsc.subcore_barrier()

    def _wr_blk(w, _):
        b = w * 16 + s

        @pl.when(b < nblocks)
        def _():
            rs = pl.ds(b * ROWB, ROWB)
            pltpu.sync_copy(acc_s.at[rs], out_h.at[c].at[rs])
        return 0
    lax.fori_loop(0, -(-nblocks // 16), _wr_blk, 0)


def _sc_edge(kvh, qh, weh, idx3, n_dst):
    nchunks = idx3.shape[0] - 1
    n_src = kvh.shape[0] // 2
    n_q = qh.shape[0] // 2
    nblocks = -(-n_dst // ROWB)
    nj = -(-nchunks // 16)
    assert n_q == n_dst

    mesh = plsc.VectorSubcoreMesh(core_axis_name="c", subcore_axis_name="s",
                                  num_cores=2, num_subcores=16)
    body = functools.partial(_sc_edge_body, nchunks, nblocks, n_src, n_q, nj)
    kfn = pl.kernel(
        body,
        out_type=jax.ShapeDtypeStruct((2, nblocks * ROWB, ACCW), jnp.float32),
        mesh=mesh,
        compiler_params=pltpu.CompilerParams(use_tc_tiling_on_sc=False,
                                             needs_layout_passes=False),
        scratch_types=[
            pltpu.VMEM_SHARED((nblocks * ROWB, ACCW), jnp.float32),
            pltpu.VMEM((2 * ED, C), jnp.float32),
            pltpu.VMEM((CH, 2 * C), jnp.float32),
            pltpu.VMEM((CH, 2 * C), jnp.float32),
            pltpu.VMEM((CH, C), jnp.float32),
            pltpu.VMEM((CH, C), jnp.float32),
            pltpu.VMEM((CH, ACCW), jnp.float32),
            pltpu.VMEM((CH, ACCW), jnp.float32),
            pltpu.VMEM((4, CH), jnp.int32),
            pltpu.VMEM((4, CH), jnp.int32),
            pltpu.VMEM((CH,), jnp.int32),
            pltpu.VMEM((CH,), jnp.int32),
            pltpu.VMEM((2, 3, CH), jnp.int32),
            pltpu.VMEM((2, 3, CH), jnp.int32),
            pltpu.SemaphoreType.DMA,
            pltpu.SemaphoreType.DMA,
            pltpu.SemaphoreType.DMA,
            pltpu.SemaphoreType.DMA,
            pltpu.SemaphoreType.DMA,
            pltpu.SemaphoreType.DMA,
            pltpu.SemaphoreType.DMA,
            pltpu.SemaphoreType.DMA,
        ],
    )
    return kfn(kvh, qh, weh, idx3)


def _gate1_body(acc_ref, xr_ref, ga_ref, gb_ref, lg_ref, lb_ref, o_ref):
    a0 = acc_ref[0]
    a1 = acc_ref[1]
    o0 = a0[:, :64] / (a0[:, 64:65] + 1e-16)
    o1 = a1[:, :64] / (a1[:, 64:65] + 1e-16)
    out = jnp.concatenate([o0, o1], axis=1)
    xr = xr_ref[...]
    beta = jax.nn.sigmoid(
        jnp.sum(xr * ga_ref[...] + out * gb_ref[...], axis=1, keepdims=True))
    h = beta * xr + (1.0 - beta) * out
    mu = jnp.mean(h, axis=1, keepdims=True)
    var = jnp.mean(jnp.square(h - mu), axis=1, keepdims=True)
    h = (h - mu) * jax.lax.rsqrt(var + 1e-5) * lg_ref[...] + lb_ref[...]
    o_ref[...] = jnp.maximum(h, 0.0)


def _gate1(acc, xr, ga, gb, lg, lb, block_rows=1000):
    n = xr.shape[0]
    vec = lambda v: v.reshape(1, -1)
    return pl.pallas_call(
        _gate1_body,
        grid=(n // block_rows,),
        in_specs=[
            pl.BlockSpec((2, block_rows, ACCW), lambda i: (0, i, 0)),
            pl.BlockSpec((block_rows, D), lambda i: (i, 0)),
            pl.BlockSpec((1, D), lambda i: (0, 0)),
            pl.BlockSpec((1, D), lambda i: (0, 0)),
            pl.BlockSpec((1, D), lambda i: (0, 0)),
            pl.BlockSpec((1, D), lambda i: (0, 0)),
        ],
        out_specs=pl.BlockSpec((block_rows, D), lambda i: (i, 0)),
        out_shape=jax.ShapeDtypeStruct((n, D), jnp.float32),
    )(acc, xr, vec(ga), vec(gb), vec(lg), vec(lb))


def _gate2_body(acc_ref, xr_ref, ga_ref, gb_ref, o_ref):
    a0 = acc_ref[0]
    a1 = acc_ref[1]
    o0 = a0[:, :64] / (a0[:, 64:65] + 1e-16)
    o1 = a1[:, :64] / (a1[:, 64:65] + 1e-16)
    out = 0.5 * (o0 + o1)
    xr = xr_ref[...]
    beta = jax.nn.sigmoid(
        jnp.sum(xr * ga_ref[...] + out * gb_ref[...], axis=1, keepdims=True))
    o = beta * xr + (1.0 - beta) * out
    m = jnp.max(o, axis=1, keepdims=True)
    lse = m + jnp.log(jnp.sum(jnp.exp(o - m), axis=1, keepdims=True))
    o_ref[...] = o - lse


def _gate2(acc, xr, ga, gb, block_rows=1000):
    n = xr.shape[0]
    vec = lambda v: v.reshape(1, -1)
    return pl.pallas_call(
        _gate2_body,
        grid=(n // block_rows,),
        in_specs=[
            pl.BlockSpec((2, block_rows, ACCW), lambda i: (0, i, 0)),
            pl.BlockSpec((block_rows, C), lambda i: (i, 0)),
            pl.BlockSpec((1, C), lambda i: (0, 0)),
            pl.BlockSpec((1, C), lambda i: (0, 0)),
        ],
        out_specs=pl.BlockSpec((block_rows, C), lambda i: (i, 0)),
        out_shape=jax.ShapeDtypeStruct((n, C), jnp.float32),
    )(acc, xr, vec(ga), vec(gb))


def _edge_blocks(src, dst, eid):
    e = src.shape[0]
    blk = jnp.stack([src.reshape(e // CH, CH), dst.reshape(e // CH, CH),
                     eid.reshape(e // CH, CH)], axis=1)
    return jnp.pad(blk, ((0, 1), (0, 0), (0, 0)))


def _weh(We):
    return We.reshape(ED, 2, C).transpose(1, 0, 2).reshape(2 * ED, C)


def kernel(x, src1, dst1, eid1, src2, dst2, eid2,
           Wq1, bq1, Wk1, bk1, Wv1, bv1, We1, Ws1, bs1, Wb1, ln_g, ln_b,
           Wq2, bq2, Wk2, bk2, Wv2, bv2, We2, Ws2, bs2, Wb2):
    kvh1 = _proj_kv(x, Wk1, bk1, Wv1, bv1).reshape(2 * N0, 2 * C)
    qh1, xr1 = _proj_qxr(x[:N1], Wq1, bq1, Ws1, bs1)
    qh1 = qh1.reshape(2 * N1, C)
    acc1 = _sc_edge(kvh1, qh1, _weh(We1),
                    _edge_blocks(src1, dst1, eid1[:, 0]), N1)
    ga1 = Wb1[:D, 0] + Wb1[2 * D:, 0]
    gb1 = Wb1[D:2 * D, 0] - Wb1[2 * D:, 0]
    h = _gate1(acc1, xr1, ga1, gb1, ln_g, ln_b)

    kvh2 = _proj_kv(h, Wk2, bk2, Wv2, bv2).reshape(2 * N1, 2 * C)
    qh2, xr2 = _proj_qxr(h[:N2], Wq2, bq2, Ws2, bs2)
    qh2 = qh2.reshape(2 * N2, C)
    acc2 = _sc_edge(kvh2, qh2, _weh(We2),
                    _edge_blocks(src2, dst2, eid2[:, 0]), N2)
    ga2 = Wb2[:C, 0] + Wb2[2 * C:, 0]
    gb2 = Wb2[C:2 * C, 0] - Wb2[2 * C:, 0]
    return _gate2(acc2, xr2, ga2, gb2)


# speedup vs baseline: 1.1620x; 1.1532x over previous
import functools

import jax
import jax.numpy as jnp
from jax import lax
from jax.experimental import pallas as pl
from jax.experimental.pallas import tpu as pltpu
from jax.experimental.pallas import tpu_sc as plsc

N0, N1, N2 = 50000, 10000, 2000
E1, E2 = 160000, 32000
D = 128
H = 2
C = 64
ED = 23

CH = 128
ACCW = 80
ROWB = 128
MMB = 1000


def _kv_body(x_ref, w_ref, b_ref, o_ref):
    x = x_ref[...]
    o_ref[0] = jnp.dot(x, w_ref[0], preferred_element_type=jnp.float32) + b_ref[0]
    o_ref[1] = jnp.dot(x, w_ref[1], preferred_element_type=jnp.float32) + b_ref[1]


def _proj_kv(x, Wk, bk, Wv, bv):
    n = x.shape[0]
    wh = jnp.stack([
        jnp.concatenate([Wk[:, :C], Wv[:, :C]], axis=1),
        jnp.concatenate([Wk[:, C:], Wv[:, C:]], axis=1),
    ])
    bh = jnp.stack([
        jnp.concatenate([bk[:C], bv[:C]]),
        jnp.concatenate([bk[C:], bv[C:]]),
    ]).reshape(2, 1, 2 * C)
    return pl.pallas_call(
        _kv_body,
        grid=(n // MMB,),
        in_specs=[
            pl.BlockSpec((MMB, D), lambda i: (i, 0)),
            pl.BlockSpec((2, D, 2 * C), lambda i: (0, 0, 0)),
            pl.BlockSpec((2, 1, 2 * C), lambda i: (0, 0, 0)),
        ],
        out_specs=pl.BlockSpec((2, MMB, 2 * C), lambda i: (0, i, 0)),
        out_shape=jax.ShapeDtypeStruct((2, n, 2 * C), jnp.float32),
    )(x, wh, bh)


def _qxr_body(x_ref, w_ref, b_ref, q_ref, xr_ref):
    y = (jnp.dot(x_ref[...], w_ref[...], preferred_element_type=jnp.float32)
         + b_ref[...])
    q_ref[0] = y[:, :C]
    q_ref[1] = y[:, C:2 * C]
    xr_ref[...] = y[:, 2 * C:]


def _proj_qxr(x, Wq, bq, Ws, bs):
    n = x.shape[0]
    dxr = Ws.shape[1]
    wc = jnp.concatenate([Wq[:, :C], Wq[:, C:], Ws], axis=1)
    bc = jnp.concatenate([bq[:C], bq[C:], bs]).reshape(1, -1)
    dout = 2 * C + dxr
    return pl.pallas_call(
        _qxr_body,
        grid=(n // MMB,),
        in_specs=[
            pl.BlockSpec((MMB, D), lambda i: (i, 0)),
            pl.BlockSpec((D, dout), lambda i: (0, 0)),
            pl.BlockSpec((1, dout), lambda i: (0, 0)),
        ],
        out_specs=[
            pl.BlockSpec((2, MMB, C), lambda i: (0, i, 0)),
            pl.BlockSpec((MMB, dxr), lambda i: (i, 0)),
        ],
        out_shape=[
            jax.ShapeDtypeStruct((2, n, C), jnp.float32),
            jax.ShapeDtypeStruct((n, dxr), jnp.float32),
        ],
    )(x, wc, bc)


def _sc_edge_body(nchunks, nblocks, n_src, n_q, nj,
                  kv_h, q_h, we_h, idx3_h, out_h,
                  acc_s, wevm,
                  kvb0, kvb1, qrb0, qrb1, srow0, srow1,
                  idxb0, idxb1, sidx0, sidx1, ipa, ipb,
                  sem_kv0, sem_kv1, sem_q0, sem_q1, sem_s0, sem_s1,
                  sem_ia, sem_ib):
    c = lax.axis_index("c")
    s = lax.axis_index("s")
    zero = jnp.zeros((16,), jnp.float32)
    lane = lax.iota(jnp.int32, 16)
    c_src = c * n_src
    c_dst = c * n_q
    c_eid = c * ED
    base = s * nj
    B0 = (kvb0, qrb0, srow0, idxb0, sidx0, sem_kv0, sem_q0, sem_s0)
    B1 = (kvb1, qrb1, srow1, idxb1, sidx1, sem_kv1, sem_q1, sem_s1)
    IP = ((ipa, sem_ia), (ipb, sem_ib))

    pltpu.sync_copy(we_h, wevm)

    def _zero_row(r, _):
        for j in range(ACCW // 16):
            srow0[r, pl.ds(16 * j, 16)] = zero
        return 0
    lax.fori_loop(0, ROWB, _zero_row, 0)

    def _zero_blk(w, _):
        b = w * 16 + s

        @pl.when(b < nblocks)
        def _():
            pltpu.sync_copy(srow0, acc_s.at[pl.ds(b * ROWB, ROWB)])
        return 0
    lax.fori_loop(0, -(-nblocks // 16), _zero_blk, 0)
    plsc.subcore_barrier()

    def _issue_pair(q2, ip, sem):
        cid = base + 2 * q2

        @pl.when((2 * q2 < nj) & (cid < nchunks))
        def _():
            pltpu.make_async_copy(idx3_h.at[pl.ds(cid, 2)], ip, sem).start()

    def _stage(j, pr, ipsel, B):
        kvb, qrb, srow, idxb, sidx, sem_kv, sem_q, sem_s = B
        ip, sem_i = IP[ipsel]
        cid = base + j

        @pl.when((j < nj) & (cid < nchunks))
        def _():
            if pr == 0:
                pltpu.make_async_copy(
                    idx3_h.at[pl.ds(cid, 2)], ip, sem_i).wait()
            for r in range(CH // 16):
                sl = pl.ds(16 * r, 16)
                dstr = ip[pr, 1, sl]
                idxb[0, sl] = ip[pr, 0, sl] + c_src
                idxb[1, sl] = dstr
                idxb[2, sl] = dstr + c_dst
                idxb[3, sl] = ip[pr, 2, sl] + c_eid
            pltpu.make_async_copy(kv_h.at[idxb.at[0]], kvb, sem_kv).start()
            pltpu.make_async_copy(q_h.at[idxb.at[2]], qrb, sem_q).start()
        if pr == 1:
            _issue_pair((j - 1) // 2 + 2, ip, sem_i)

    def _wait_gathers(j, B):
        kvb, qrb, srow, idxb, sidx, sem_kv, sem_q, sem_s = B
        cid = base + j

        @pl.when((j < nj) & (cid < nchunks))
        def _():
            pltpu.make_async_copy(kv_h.at[idxb.at[0]], kvb, sem_kv).wait()
            pltpu.make_async_copy(q_h.at[idxb.at[2]], qrb, sem_q).wait()

    def _wait_scatter(j, B):
        kvb, qrb, srow, idxb, sidx, sem_kv, sem_q, sem_s = B
        cid = base + j

        @pl.when((j >= 0) & (j < nj) & (cid < nchunks))
        def _():
            pltpu.make_async_copy(srow, acc_s.at[sidx], sem_s).wait()

    def _compute_scatter(j, B):
        kvb, qrb, srow, idxb, sidx, sem_kv, sem_q, sem_s = B
        cid = base + j

        @pl.when((j < nj) & (cid < nchunks))
        def _():
            @plsc.parallel_loop(0, CH, unroll=16)
            def _edge(ei):
                eid16 = plsc.load_gather(
                    idxb.at[3], [jnp.full((16,), ei, jnp.int32)])
                er = [plsc.load_gather(wevm, [eid16, lane + 16 * jj])
                      for jj in range(4)]
                acc = zero
                for jj in range(4):
                    sl = pl.ds(16 * jj, 16)
                    acc = acc + (kvb[ei, sl] + er[jj]) * qrb[ei, sl]
                a = jnp.sum(acc) * 0.125
                ex = jnp.exp(jnp.full((16,), a, jnp.float32))
                srow[ei, pl.ds(64, 16)] = jnp.where(lane == 0, ex, zero)
                for jj in range(4):
                    sl = pl.ds(16 * jj, 16)
                    slv = pl.ds(64 + 16 * jj, 16)
                    srow[ei, sl] = (kvb[ei, slv] + er[jj]) * ex

            for r in range(CH // 16):
                sl = pl.ds(16 * r, 16)
                sidx[sl] = idxb[1, sl]
            pltpu.make_async_copy(srow, acc_s.at[sidx], sem_s).start(add=True)

    T = -(-nj // 4)
    _issue_pair(0, ipa, sem_ia)
    _issue_pair(1, ipb, sem_ib)
    _stage(0, 0, 0, B0)

    def _pipe(t, _):
        j0 = 4 * t
        for k in range(4):
            j = j0 + k
            bufs = (B0, B1)[k % 2]
            nbufs = (B1, B0)[k % 2]
            _stage(j + 1, (k + 1) % 2, ((k + 1) // 2) % 2, nbufs)
            _wait_gathers(j, bufs)
            _wait_scatter(j - 2, bufs)
            _compute_scatter(j, bufs)
        return 0

    lax.fori_loop(0, T, _pipe, 0)
    _wait_scatter(4 * T - 2, B0)
    _wait_scatter(4 * T - 1, B1)
    plsc.subcore_barrier()

    def _wr_blk(w, _):
        b = w * 16 + s

        @pl.when(b < nblocks)
        def _():
            rs = pl.ds(b * ROWB, ROWB)
            pltpu.sync_copy(acc_s.at[rs], out_h.at[c].at[rs])
        return 0
    lax.fori_loop(0, -(-nblocks // 16), _wr_blk, 0)


def _sc_edge(kvh, qh, weh, idx3, n_dst):
    nchunks = idx3.shape[0] - 1
    n_src = kvh.shape[0] // 2
    n_q = qh.shape[0] // 2
    nblocks = -(-n_dst // ROWB)
    nj = -(-nchunks // 16)
    assert n_q == n_dst

    mesh = plsc.VectorSubcoreMesh(core_axis_name="c", subcore_axis_name="s",
                                  num_cores=2, num_subcores=16)
    body = functools.partial(_sc_edge_body, nchunks, nblocks, n_src, n_q, nj)
    kfn = pl.kernel(
        body,
        out_type=jax.ShapeDtypeStruct((2, nblocks * ROWB, ACCW), jnp.float32),
        mesh=mesh,
        compiler_params=pltpu.CompilerParams(use_tc_tiling_on_sc=False,
                                             needs_layout_passes=False),
        scratch_types=[
            pltpu.VMEM_SHARED((nblocks * ROWB, ACCW), jnp.float32),
            pltpu.VMEM((2 * ED, C), jnp.float32),
            pltpu.VMEM((CH, 2 * C), jnp.float32),
            pltpu.VMEM((CH, 2 * C), jnp.float32),
            pltpu.VMEM((CH, C), jnp.float32),
            pltpu.VMEM((CH, C), jnp.float32),
            pltpu.VMEM((CH, ACCW), jnp.float32),
            pltpu.VMEM((CH, ACCW), jnp.float32),
            pltpu.VMEM((4, CH), jnp.int32),
            pltpu.VMEM((4, CH), jnp.int32),
            pltpu.VMEM((CH,), jnp.int32),
            pltpu.VMEM((CH,), jnp.int32),
            pltpu.VMEM((2, 3, CH), jnp.int32),
            pltpu.VMEM((2, 3, CH), jnp.int32),
            pltpu.SemaphoreType.DMA,
            pltpu.SemaphoreType.DMA,
            pltpu.SemaphoreType.DMA,
            pltpu.SemaphoreType.DMA,
            pltpu.SemaphoreType.DMA,
            pltpu.SemaphoreType.DMA,
            pltpu.SemaphoreType.DMA,
            pltpu.SemaphoreType.DMA,
        ],
    )
    return kfn(kvh, qh, weh, idx3)


def _gate1_body(acc_ref, xr_ref, ga_ref, gb_ref, lg_ref, lb_ref, o_ref):
    a0 = acc_ref[0]
    a1 = acc_ref[1]
    o0 = a0[:, :64] / (a0[:, 64:65] + 1e-16)
    o1 = a1[:, :64] / (a1[:, 64:65] + 1e-16)
    out = jnp.concatenate([o0, o1], axis=1)
    xr = xr_ref[...]
    beta = jax.nn.sigmoid(
        jnp.sum(xr * ga_ref[...] + out * gb_ref[...], axis=1, keepdims=True))
    h = beta * xr + (1.0 - beta) * out
    mu = jnp.mean(h, axis=1, keepdims=True)
    var = jnp.mean(jnp.square(h - mu), axis=1, keepdims=True)
    h = (h - mu) * jax.lax.rsqrt(var + 1e-5) * lg_ref[...] + lb_ref[...]
    o_ref[...] = jnp.maximum(h, 0.0)


def _gate1(acc, xr, ga, gb, lg, lb, block_rows=1000):
    n = xr.shape[0]
    vec = lambda v: v.reshape(1, -1)
    return pl.pallas_call(
        _gate1_body,
        grid=(n // block_rows,),
        in_specs=[
            pl.BlockSpec((2, block_rows, ACCW), lambda i: (0, i, 0)),
            pl.BlockSpec((block_rows, D), lambda i: (i, 0)),
            pl.BlockSpec((1, D), lambda i: (0, 0)),
            pl.BlockSpec((1, D), lambda i: (0, 0)),
            pl.BlockSpec((1, D), lambda i: (0, 0)),
            pl.BlockSpec((1, D), lambda i: (0, 0)),
        ],
        out_specs=pl.BlockSpec((block_rows, D), lambda i: (i, 0)),
        out_shape=jax.ShapeDtypeStruct((n, D), jnp.float32),
    )(acc, xr, vec(ga), vec(gb), vec(lg), vec(lb))


def _gate2_body(acc_ref, xr_ref, ga_ref, gb_ref, o_ref):
    a0 = acc_ref[0]
    a1 = acc_ref[1]
    o0 = a0[:, :64] / (a0[:, 64:65] + 1e-16)
    o1 = a1[:, :64] / (a1[:, 64:65] + 1e-16)
    out = 0.5 * (o0 + o1)
    xr = xr_ref[...]
    beta = jax.nn.sigmoid(
        jnp.sum(xr * ga_ref[...] + out * gb_ref[...], axis=1, keepdims=True))
    o = beta * xr + (1.0 - beta) * out
    m = jnp.max(o, axis=1, keepdims=True)
    lse = m + jnp.log(jnp.sum(jnp.exp(o - m), axis=1, keepdims=True))
    o_ref[...] = o - lse


def _gate2(acc, xr, ga, gb, block_rows=1000):
    n = xr.shape[0]
    vec = lambda v: v.reshape(1, -1)
    return pl.pallas_call(
        _gate2_body,
        grid=(n // block_rows,),
        in_specs=[
            pl.BlockSpec((2, block_rows, ACCW), lambda i: (0, i, 0)),
            pl.BlockSpec((block_rows, C), lambda i: (i, 0)),
            pl.BlockSpec((1, C), lambda i: (0, 0)),
            pl.BlockSpec((1, C), lambda i: (0, 0)),
        ],
        out_specs=pl.BlockSpec((block_rows, C), lambda i: (i, 0)),
        out_shape=jax.ShapeDtypeStruct((n, C), jnp.float32),
    )(acc, xr, vec(ga), vec(gb))


def _edge_blocks(src, dst, eid):
    e = src.shape[0]
    blk = jnp.stack([src.reshape(e // CH, CH), dst.reshape(e // CH, CH),
                     eid.reshape(e // CH, CH)], axis=1)
    return jnp.pad(blk, ((0, 1), (0, 0), (0, 0)))


def _weh(We):
    return We.reshape(ED, 2, C).transpose(1, 0, 2).reshape(2 * ED, C)


def kernel(x, src1, dst1, eid1, src2, dst2, eid2,
           Wq1, bq1, Wk1, bk1, Wv1, bv1, We1, Ws1, bs1, Wb1, ln_g, ln_b,
           Wq2, bq2, Wk2, bk2, Wv2, bv2, We2, Ws2, bs2, Wb2):
    kvh1 = _proj_kv(x, Wk1, bk1, Wv1, bv1).reshape(2 * N0, 2 * C)
    qh1, xr1 = _proj_qxr(x[:N1], Wq1, bq1, Ws1, bs1)
    qh1 = qh1.reshape(2 * N1, C)
    acc1 = _sc_edge(kvh1, qh1, _weh(We1),
                    _edge_blocks(src1, dst1, eid1[:, 0]), N1)
    ga1 = Wb1[:D, 0] + Wb1[2 * D:, 0]
    gb1 = Wb1[D:2 * D, 0] - Wb1[2 * D:, 0]
    h = _gate1(acc1, xr1, ga1, gb1, ln_g, ln_b)

    kvh2 = _proj_kv(h, Wk2, bk2, Wv2, bv2).reshape(2 * N1, 2 * C)
    qh2, xr2 = _proj_qxr(h[:N2], Wq2, bq2, Ws2, bs2)
    qh2 = qh2.reshape(2 * N2, C)
    acc2 = _sc_edge(kvh2, qh2, _weh(We2),
                    _edge_blocks(src2, dst2, eid2[:, 0]), N2)
    ga2 = Wb2[:C, 0] + Wb2[2 * C:, 0]
    gb2 = Wb2[C:2 * C, 0] - Wb2[2 * C:, 0]
    return _gate2(acc2, xr2, ga2, gb2)


# speedup vs baseline: 1.2321x; 1.0603x over previous
import functools

import jax
import jax.numpy as jnp
from jax import lax
from jax.experimental import pallas as pl
from jax.experimental.pallas import tpu as pltpu
from jax.experimental.pallas import tpu_sc as plsc

N0, N1, N2 = 50000, 10000, 2000
E1, E2 = 160000, 32000
D = 128
H = 2
C = 64
ED = 23

CH = 128
ACCW = 80
ROWB = 128
MMB = 2000


def _kv_body(x_ref, w_ref, b_ref, o_ref):
    x = x_ref[...]
    o_ref[0] = jnp.dot(x, w_ref[0], preferred_element_type=jnp.float32) + b_ref[0]
    o_ref[1] = jnp.dot(x, w_ref[1], preferred_element_type=jnp.float32) + b_ref[1]


def _proj_kv(x, Wk, bk, Wv, bv):
    n = x.shape[0]
    wh = jnp.stack([
        jnp.concatenate([Wk[:, :C], Wv[:, :C]], axis=1),
        jnp.concatenate([Wk[:, C:], Wv[:, C:]], axis=1),
    ])
    bh = jnp.stack([
        jnp.concatenate([bk[:C], bv[:C]]),
        jnp.concatenate([bk[C:], bv[C:]]),
    ]).reshape(2, 1, 2 * C)
    return pl.pallas_call(
        _kv_body,
        grid=(n // MMB,),
        in_specs=[
            pl.BlockSpec((MMB, D), lambda i: (i, 0)),
            pl.BlockSpec((2, D, 2 * C), lambda i: (0, 0, 0)),
            pl.BlockSpec((2, 1, 2 * C), lambda i: (0, 0, 0)),
        ],
        out_specs=pl.BlockSpec((2, MMB, 2 * C), lambda i: (0, i, 0)),
        out_shape=jax.ShapeDtypeStruct((2, n, 2 * C), jnp.float32),
    )(x, wh, bh)


def _qxr_body(x_ref, w_ref, b_ref, q_ref, xr_ref):
    y = (jnp.dot(x_ref[...], w_ref[...], preferred_element_type=jnp.float32)
         + b_ref[...])
    q_ref[0] = y[:, :C]
    q_ref[1] = y[:, C:2 * C]
    xr_ref[...] = y[:, 2 * C:]


def _proj_qxr(x, Wq, bq, Ws, bs):
    n = x.shape[0]
    dxr = Ws.shape[1]
    wc = jnp.concatenate([Wq[:, :C], Wq[:, C:], Ws], axis=1)
    bc = jnp.concatenate([bq[:C], bq[C:], bs]).reshape(1, -1)
    dout = 2 * C + dxr
    return pl.pallas_call(
        _qxr_body,
        grid=(n // MMB,),
        in_specs=[
            pl.BlockSpec((MMB, D), lambda i: (i, 0)),
            pl.BlockSpec((D, dout), lambda i: (0, 0)),
            pl.BlockSpec((1, dout), lambda i: (0, 0)),
        ],
        out_specs=[
            pl.BlockSpec((2, MMB, C), lambda i: (0, i, 0)),
            pl.BlockSpec((MMB, dxr), lambda i: (i, 0)),
        ],
        out_shape=[
            jax.ShapeDtypeStruct((2, n, C), jnp.float32),
            jax.ShapeDtypeStruct((n, dxr), jnp.float32),
        ],
    )(x, wc, bc)


def _sc_edge_body(nchunks, nblocks, n_src, n_q, nj,
                  kv_h, q_h, we_h, idx3_h, out_h,
                  acc_s, wevm,
                  kvb0, kvb1, qrb0, qrb1, srow0, srow1,
                  idxb0, idxb1, sidx0, sidx1, ipa, ipb,
                  sem_kv0, sem_kv1, sem_q0, sem_q1, sem_s0, sem_s1,
                  sem_ia, sem_ib):
    c = lax.axis_index("c")
    s = lax.axis_index("s")
    zero = jnp.zeros((16,), jnp.float32)
    lane = lax.iota(jnp.int32, 16)
    c_src = c * n_src
    c_dst = c * n_q
    c_eid = c * ED
    base = s * nj
    B0 = (kvb0, qrb0, srow0, idxb0, sidx0, sem_kv0, sem_q0, sem_s0)
    B1 = (kvb1, qrb1, srow1, idxb1, sidx1, sem_kv1, sem_q1, sem_s1)
    IP = ((ipa, sem_ia), (ipb, sem_ib))

    pltpu.sync_copy(we_h, wevm)

    def _zero_row(r, _):
        for j in range(ACCW // 16):
            srow0[r, pl.ds(16 * j, 16)] = zero
        return 0
    lax.fori_loop(0, ROWB, _zero_row, 0)

    def _zero_blk(w, _):
        b = w * 16 + s

        @pl.when(b < nblocks)
        def _():
            pltpu.sync_copy(srow0, acc_s.at[pl.ds(b * ROWB, ROWB)])
        return 0
    lax.fori_loop(0, -(-nblocks // 16), _zero_blk, 0)
    plsc.subcore_barrier()

    def _issue_pair(q2, ip, sem):
        cid = base + 2 * q2

        @pl.when((2 * q2 < nj) & (cid < nchunks))
        def _():
            pltpu.make_async_copy(idx3_h.at[pl.ds(cid, 2)], ip, sem).start()

    def _stage(j, pr, ipsel, B):
        kvb, qrb, srow, idxb, sidx, sem_kv, sem_q, sem_s = B
        ip, sem_i = IP[ipsel]
        cid = base + j

        @pl.when((j < nj) & (cid < nchunks))
        def _():
            if pr == 0:
                pltpu.make_async_copy(
                    idx3_h.at[pl.ds(cid, 2)], ip, sem_i).wait()
            for r in range(CH // 16):
                sl = pl.ds(16 * r, 16)
                dstr = ip[pr, 1, sl]
                idxb[0, sl] = ip[pr, 0, sl] + c_src
                idxb[1, sl] = dstr
                idxb[2, sl] = dstr + c_dst
                idxb[3, sl] = ip[pr, 2, sl] + c_eid
            pltpu.make_async_copy(kv_h.at[idxb.at[0]], kvb, sem_kv).start()
            pltpu.make_async_copy(q_h.at[idxb.at[2]], qrb, sem_q).start()
        if pr == 1:
            _issue_pair((j - 1) // 2 + 2, ip, sem_i)

    def _wait_gathers(j, B):
        kvb, qrb, srow, idxb, sidx, sem_kv, sem_q, sem_s = B
        cid = base + j

        @pl.when((j < nj) & (cid < nchunks))
        def _():
            pltpu.make_async_copy(kv_h.at[idxb.at[0]], kvb, sem_kv).wait()
            pltpu.make_async_copy(q_h.at[idxb.at[2]], qrb, sem_q).wait()

    def _wait_scatter(j, B):
        kvb, qrb, srow, idxb, sidx, sem_kv, sem_q, sem_s = B
        cid = base + j

        @pl.when((j >= 0) & (j < nj) & (cid < nchunks))
        def _():
            pltpu.make_async_copy(srow, acc_s.at[sidx], sem_s).wait()

    def _compute_scatter(j, B):
        kvb, qrb, srow, idxb, sidx, sem_kv, sem_q, sem_s = B
        cid = base + j

        @pl.when((j < nj) & (cid < nchunks))
        def _():
            @plsc.parallel_loop(0, CH, unroll=16)
            def _edge(ei):
                eid16 = plsc.load_gather(
                    idxb.at[3], [jnp.full((16,), ei, jnp.int32)])
                er = [plsc.load_gather(wevm, [eid16, lane + 16 * jj])
                      for jj in range(4)]
                acc = zero
                for jj in range(4):
                    sl = pl.ds(16 * jj, 16)
                    acc = acc + (kvb[ei, sl] + er[jj]) * qrb[ei, sl]
                a = jnp.sum(acc) * 0.125
                ex = jnp.exp(jnp.full((16,), a, jnp.float32))
                srow[ei, pl.ds(64, 16)] = jnp.where(lane == 0, ex, zero)
                for jj in range(4):
                    sl = pl.ds(16 * jj, 16)
                    slv = pl.ds(64 + 16 * jj, 16)
                    srow[ei, sl] = (kvb[ei, slv] + er[jj]) * ex

            for r in range(CH // 16):
                sl = pl.ds(16 * r, 16)
                sidx[sl] = idxb[1, sl]
            pltpu.make_async_copy(srow, acc_s.at[sidx], sem_s).start(add=True)

    T = -(-nj // 4)
    _issue_pair(0, ipa, sem_ia)
    _issue_pair(1, ipb, sem_ib)
    _stage(0, 0, 0, B0)

    def _pipe(t, _):
        j0 = 4 * t
        for k in range(4):
            j = j0 + k
            bufs = (B0, B1)[k % 2]
            nbufs = (B1, B0)[k % 2]
            _stage(j + 1, (k + 1) % 2, ((k + 1) // 2) % 2, nbufs)
            _wait_gathers(j, bufs)
            _wait_scatter(j - 2, bufs)
            _compute_scatter(j, bufs)
        return 0

    lax.fori_loop(0, T, _pipe, 0)
    _wait_scatter(4 * T - 2, B0)
    _wait_scatter(4 * T - 1, B1)
    plsc.subcore_barrier()

    def _wr_blk(w, _):
        b = w * 16 + s

        @pl.when(b < nblocks)
        def _():
            rs = pl.ds(b * ROWB, ROWB)
            pltpu.sync_copy(acc_s.at[rs], out_h.at[c].at[rs])
        return 0
    lax.fori_loop(0, -(-nblocks // 16), _wr_blk, 0)


def _sc_edge(kvh, qh, weh, idx3, n_dst):
    nchunks = idx3.shape[0] - 1
    n_src = kvh.shape[0] // 2
    n_q = qh.shape[0] // 2
    nblocks = -(-n_dst // ROWB)
    nj = -(-nchunks // 16)
    assert n_q == n_dst

    mesh = plsc.VectorSubcoreMesh(core_axis_name="c", subcore_axis_name="s",
                                  num_cores=2, num_subcores=16)
    body = functools.partial(_sc_edge_body, nchunks, nblocks, n_src, n_q, nj)
    kfn = pl.kernel(
        body,
        out_type=jax.ShapeDtypeStruct((2, nblocks * ROWB, ACCW), jnp.float32),
        mesh=mesh,
        compiler_params=pltpu.CompilerParams(use_tc_tiling_on_sc=False,
                                             needs_layout_passes=False),
        scratch_types=[
            pltpu.VMEM_SHARED((nblocks * ROWB, ACCW), jnp.float32),
            pltpu.VMEM((2 * ED, C), jnp.float32),
            pltpu.VMEM((CH, 2 * C), jnp.float32),
            pltpu.VMEM((CH, 2 * C), jnp.float32),
            pltpu.VMEM((CH, C), jnp.float32),
            pltpu.VMEM((CH, C), jnp.float32),
            pltpu.VMEM((CH, ACCW), jnp.float32),
            pltpu.VMEM((CH, ACCW), jnp.float32),
            pltpu.VMEM((4, CH), jnp.int32),
            pltpu.VMEM((4, CH), jnp.int32),
            pltpu.VMEM((CH,), jnp.int32),
            pltpu.VMEM((CH,), jnp.int32),
            pltpu.VMEM((2, 3, CH), jnp.int32),
            pltpu.VMEM((2, 3, CH), jnp.int32),
            pltpu.SemaphoreType.DMA,
            pltpu.SemaphoreType.DMA,
            pltpu.SemaphoreType.DMA,
            pltpu.SemaphoreType.DMA,
            pltpu.SemaphoreType.DMA,
            pltpu.SemaphoreType.DMA,
            pltpu.SemaphoreType.DMA,
            pltpu.SemaphoreType.DMA,
        ],
    )
    return kfn(kvh, qh, weh, idx3)


def _gate1_body(acc_ref, xr_ref, ga_ref, gb_ref, lg_ref, lb_ref, o_ref):
    a0 = acc_ref[0]
    a1 = acc_ref[1]
    o0 = a0[:, :64] / (a0[:, 64:65] + 1e-16)
    o1 = a1[:, :64] / (a1[:, 64:65] + 1e-16)
    out = jnp.concatenate([o0, o1], axis=1)
    xr = xr_ref[...]
    beta = jax.nn.sigmoid(
        jnp.sum(xr * ga_ref[...] + out * gb_ref[...], axis=1, keepdims=True))
    h = beta * xr + (1.0 - beta) * out
    mu = jnp.mean(h, axis=1, keepdims=True)
    var = jnp.mean(jnp.square(h - mu), axis=1, keepdims=True)
    h = (h - mu) * jax.lax.rsqrt(var + 1e-5) * lg_ref[...] + lb_ref[...]
    o_ref[...] = jnp.maximum(h, 0.0)


def _gate1(acc, xr, ga, gb, lg, lb, block_rows=1000):
    n = xr.shape[0]
    vec = lambda v: v.reshape(1, -1)
    return pl.pallas_call(
        _gate1_body,
        grid=(n // block_rows,),
        in_specs=[
            pl.BlockSpec((2, block_rows, ACCW), lambda i: (0, i, 0)),
            pl.BlockSpec((block_rows, D), lambda i: (i, 0)),
            pl.BlockSpec((1, D), lambda i: (0, 0)),
            pl.BlockSpec((1, D), lambda i: (0, 0)),
            pl.BlockSpec((1, D), lambda i: (0, 0)),
            pl.BlockSpec((1, D), lambda i: (0, 0)),
        ],
        out_specs=pl.BlockSpec((block_rows, D), lambda i: (i, 0)),
        out_shape=jax.ShapeDtypeStruct((n, D), jnp.float32),
    )(acc, xr, vec(ga), vec(gb), vec(lg), vec(lb))


def _gate2_body(acc_ref, xr_ref, ga_ref, gb_ref, o_ref):
    a0 = acc_ref[0]
    a1 = acc_ref[1]
    o0 = a0[:, :64] / (a0[:, 64:65] + 1e-16)
    o1 = a1[:, :64] / (a1[:, 64:65] + 1e-16)
    out = 0.5 * (o0 + o1)
    xr = xr_ref[...]
    beta = jax.nn.sigmoid(
        jnp.sum(xr * ga_ref[...] + out * gb_ref[...], axis=1, keepdims=True))
    o = beta * xr + (1.0 - beta) * out
    m = jnp.max(o, axis=1, keepdims=True)
    lse = m + jnp.log(jnp.sum(jnp.exp(o - m), axis=1, keepdims=True))
    o_ref[...] = o - lse


def _gate2(acc, xr, ga, gb, block_rows=1000):
    n = xr.shape[0]
    vec = lambda v: v.reshape(1, -1)
    return pl.pallas_call(
        _gate2_body,
        grid=(n // block_rows,),
        in_specs=[
            pl.BlockSpec((2, block_rows, ACCW), lambda i: (0, i, 0)),
            pl.BlockSpec((block_rows, C), lambda i: (i, 0)),
            pl.BlockSpec((1, C), lambda i: (0, 0)),
            pl.BlockSpec((1, C), lambda i: (0, 0)),
        ],
        out_specs=pl.BlockSpec((block_rows, C), lambda i: (i, 0)),
        out_shape=jax.ShapeDtypeStruct((n, C), jnp.float32),
    )(acc, xr, vec(ga), vec(gb))


def _edge_blocks(src, dst, eid):
    e = src.shape[0]
    blk = jnp.stack([src.reshape(e // CH, CH), dst.reshape(e // CH, CH),
                     eid.reshape(e // CH, CH)], axis=1)
    return jnp.pad(blk, ((0, 1), (0, 0), (0, 0)))


def _weh(We):
    return We.reshape(ED, 2, C).transpose(1, 0, 2).reshape(2 * ED, C)


def kernel(x, src1, dst1, eid1, src2, dst2, eid2,
           Wq1, bq1, Wk1, bk1, Wv1, bv1, We1, Ws1, bs1, Wb1, ln_g, ln_b,
           Wq2, bq2, Wk2, bk2, Wv2, bv2, We2, Ws2, bs2, Wb2):
    kvh1 = _proj_kv(x, Wk1, bk1, Wv1, bv1).reshape(2 * N0, 2 * C)
    qh1, xr1 = _proj_qxr(x[:N1], Wq1, bq1, Ws1, bs1)
    qh1 = qh1.reshape(2 * N1, C)
    acc1 = _sc_edge(kvh1, qh1, _weh(We1),
                    _edge_blocks(src1, dst1, eid1[:, 0]), N1)
    ga1 = Wb1[:D, 0] + Wb1[2 * D:, 0]
    gb1 = Wb1[D:2 * D, 0] - Wb1[2 * D:, 0]
    h = _gate1(acc1, xr1, ga1, gb1, ln_g, ln_b)

    kvh2 = _proj_kv(h, Wk2, bk2, Wv2, bv2).reshape(2 * N1, 2 * C)
    qh2, xr2 = _proj_qxr(h[:N2], Wq2, bq2, Ws2, bs2)
    qh2 = qh2.reshape(2 * N2, C)
    acc2 = _sc_edge(kvh2, qh2, _weh(We2),
                    _edge_blocks(src2, dst2, eid2[:, 0]), N2)
    ga2 = Wb2[:C, 0] + Wb2[2 * C:, 0]
    gb2 = Wb2[C:2 * C, 0] - Wb2[2 * C:, 0]
    return _gate2(acc2, xr2, ga2, gb2)


# speedup vs baseline: 1.2715x; 1.0320x over previous
import functools

import jax
import jax.numpy as jnp
from jax import lax
from jax.experimental import pallas as pl
from jax.experimental.pallas import tpu as pltpu
from jax.experimental.pallas import tpu_sc as plsc

N0, N1, N2 = 50000, 10000, 2000
E1, E2 = 160000, 32000
D = 128
H = 2
C = 64
ED = 23

CH = 128
ACCW = 80
ROWB = 128
MMB = 2000


def _kv_body(x_ref, w_ref, b_ref, o_ref):
    x = x_ref[...]
    o_ref[0] = jnp.dot(x, w_ref[0], preferred_element_type=jnp.float32) + b_ref[0]
    o_ref[1] = jnp.dot(x, w_ref[1], preferred_element_type=jnp.float32) + b_ref[1]


def _blk(n):
    return 5000 if n % 5000 == 0 else MMB


def _proj_kv(x, Wk, bk, Wv, bv):
    n = x.shape[0]
    MB = _blk(n)
    wh = jnp.stack([
        jnp.concatenate([Wk[:, :C], Wv[:, :C]], axis=1),
        jnp.concatenate([Wk[:, C:], Wv[:, C:]], axis=1),
    ])
    bh = jnp.stack([
        jnp.concatenate([bk[:C], bv[:C]]),
        jnp.concatenate([bk[C:], bv[C:]]),
    ]).reshape(2, 1, 2 * C)
    return pl.pallas_call(
        _kv_body,
        grid=(n // MB,),
        in_specs=[
            pl.BlockSpec((MB, D), lambda i: (i, 0)),
            pl.BlockSpec((2, D, 2 * C), lambda i: (0, 0, 0)),
            pl.BlockSpec((2, 1, 2 * C), lambda i: (0, 0, 0)),
        ],
        out_specs=pl.BlockSpec((2, MB, 2 * C), lambda i: (0, i, 0)),
        out_shape=jax.ShapeDtypeStruct((2, n, 2 * C), jnp.float32),
    )(x, wh, bh)


def _qxr_body(x_ref, w_ref, b_ref, q_ref, xr_ref):
    y = (jnp.dot(x_ref[...], w_ref[...], preferred_element_type=jnp.float32)
         + b_ref[...])
    q_ref[0] = y[:, :C]
    q_ref[1] = y[:, C:2 * C]
    xr_ref[...] = y[:, 2 * C:]


def _proj_qxr(x, Wq, bq, Ws, bs):
    n = x.shape[0]
    dxr = Ws.shape[1]
    wc = jnp.concatenate([Wq[:, :C], Wq[:, C:], Ws], axis=1)
    bc = jnp.concatenate([bq[:C], bq[C:], bs]).reshape(1, -1)
    dout = 2 * C + dxr
    MB = _blk(n)
    return pl.pallas_call(
        _qxr_body,
        grid=(n // MB,),
        in_specs=[
            pl.BlockSpec((MB, D), lambda i: (i, 0)),
            pl.BlockSpec((D, dout), lambda i: (0, 0)),
            pl.BlockSpec((1, dout), lambda i: (0, 0)),
        ],
        out_specs=[
            pl.BlockSpec((2, MB, C), lambda i: (0, i, 0)),
            pl.BlockSpec((MB, dxr), lambda i: (i, 0)),
        ],
        out_shape=[
            jax.ShapeDtypeStruct((2, n, C), jnp.float32),
            jax.ShapeDtypeStruct((n, dxr), jnp.float32),
        ],
    )(x, wc, bc)


def _sc_edge_body(nchunks, nblocks, n_src, n_q, nj,
                  kv_h, q_h, we_h, idx3_h, out_h,
                  acc_s, wevm,
                  kvb0, kvb1, qrb0, qrb1, srow0, srow1,
                  idxb0, idxb1, sidx0, sidx1, ipa, ipb,
                  sem_kv0, sem_kv1, sem_q0, sem_q1, sem_s0, sem_s1,
                  sem_ia, sem_ib):
    c = lax.axis_index("c")
    s = lax.axis_index("s")
    zero = jnp.zeros((16,), jnp.float32)
    lane = lax.iota(jnp.int32, 16)
    c_src = c * n_src
    c_dst = c * n_q
    c_eid = c * ED
    base = s * nj
    B0 = (kvb0, qrb0, srow0, idxb0, sidx0, sem_kv0, sem_q0, sem_s0)
    B1 = (kvb1, qrb1, srow1, idxb1, sidx1, sem_kv1, sem_q1, sem_s1)
    IP = ((ipa, sem_ia), (ipb, sem_ib))

    pltpu.sync_copy(we_h, wevm)

    def _zero_row(r, _):
        for j in range(ACCW // 16):
            srow0[r, pl.ds(16 * j, 16)] = zero
        return 0
    lax.fori_loop(0, ROWB, _zero_row, 0)

    def _zero_blk(w, _):
        b = w * 16 + s

        @pl.when(b < nblocks)
        def _():
            pltpu.sync_copy(srow0, acc_s.at[pl.ds(b * ROWB, ROWB)])
        return 0
    lax.fori_loop(0, -(-nblocks // 16), _zero_blk, 0)
    plsc.subcore_barrier()

    def _issue_pair(q2, ip, sem):
        cid = base + 2 * q2

        @pl.when((2 * q2 < nj) & (cid < nchunks))
        def _():
            pltpu.make_async_copy(idx3_h.at[pl.ds(cid, 2)], ip, sem).start()

    def _stage(j, pr, ipsel, B):
        kvb, qrb, srow, idxb, sidx, sem_kv, sem_q, sem_s = B
        ip, sem_i = IP[ipsel]
        cid = base + j

        @pl.when((j < nj) & (cid < nchunks))
        def _():
            if pr == 0:
                pltpu.make_async_copy(
                    idx3_h.at[pl.ds(cid, 2)], ip, sem_i).wait()
            for r in range(CH // 16):
                sl = pl.ds(16 * r, 16)
                dstr = ip[pr, 1, sl]
                idxb[0, sl] = ip[pr, 0, sl] + c_src
                idxb[1, sl] = dstr
                idxb[2, sl] = dstr + c_dst
                idxb[3, sl] = ip[pr, 2, sl] + c_eid
            pltpu.make_async_copy(kv_h.at[idxb.at[0]], kvb, sem_kv).start()
            pltpu.make_async_copy(q_h.at[idxb.at[2]], qrb, sem_q).start()
        if pr == 1:
            _issue_pair((j - 1) // 2 + 2, ip, sem_i)

    def _wait_gathers(j, B):
        kvb, qrb, srow, idxb, sidx, sem_kv, sem_q, sem_s = B
        cid = base + j

        @pl.when((j < nj) & (cid < nchunks))
        def _():
            pltpu.make_async_copy(kv_h.at[idxb.at[0]], kvb, sem_kv).wait()
            pltpu.make_async_copy(q_h.at[idxb.at[2]], qrb, sem_q).wait()

    def _wait_scatter(j, B):
        kvb, qrb, srow, idxb, sidx, sem_kv, sem_q, sem_s = B
        cid = base + j

        @pl.when((j >= 0) & (j < nj) & (cid < nchunks))
        def _():
            pltpu.make_async_copy(srow, acc_s.at[sidx], sem_s).wait()

    def _compute_scatter(j, B):
        kvb, qrb, srow, idxb, sidx, sem_kv, sem_q, sem_s = B
        cid = base + j

        @pl.when((j < nj) & (cid < nchunks))
        def _():
            @plsc.parallel_loop(0, CH, unroll=16)
            def _edge(ei):
                eid16 = plsc.load_gather(
                    idxb.at[3], [jnp.full((16,), ei, jnp.int32)])
                er = [plsc.load_gather(wevm, [eid16, lane + 16 * jj])
                      for jj in range(4)]
                acc = zero
                for jj in range(4):
                    sl = pl.ds(16 * jj, 16)
                    acc = acc + (kvb[ei, sl] + er[jj]) * qrb[ei, sl]
                a = jnp.sum(acc) * 0.125
                ex = jnp.exp(jnp.full((16,), a, jnp.float32))
                srow[ei, pl.ds(64, 16)] = jnp.where(lane == 0, ex, zero)
                for jj in range(4):
                    sl = pl.ds(16 * jj, 16)
                    slv = pl.ds(64 + 16 * jj, 16)
                    srow[ei, sl] = (kvb[ei, slv] + er[jj]) * ex

            for r in range(CH // 16):
                sl = pl.ds(16 * r, 16)
                sidx[sl] = idxb[1, sl]
            pltpu.make_async_copy(srow, acc_s.at[sidx], sem_s).start(add=True)

    T = -(-nj // 4)
    _issue_pair(0, ipa, sem_ia)
    _issue_pair(1, ipb, sem_ib)
    _stage(0, 0, 0, B0)

    def _pipe(t, _):
        j0 = 4 * t
        for k in range(4):
            j = j0 + k
            bufs = (B0, B1)[k % 2]
            nbufs = (B1, B0)[k % 2]
            _stage(j + 1, (k + 1) % 2, ((k + 1) // 2) % 2, nbufs)
            _wait_gathers(j, bufs)
            _wait_scatter(j - 2, bufs)
            _compute_scatter(j, bufs)
        return 0

    lax.fori_loop(0, T, _pipe, 0)
    _wait_scatter(4 * T - 2, B0)
    _wait_scatter(4 * T - 1, B1)
    plsc.subcore_barrier()

    def _wr_blk(w, _):
        b = w * 16 + s

        @pl.when(b < nblocks)
        def _():
            rs = pl.ds(b * ROWB, ROWB)
            pltpu.sync_copy(acc_s.at[rs], out_h.at[c].at[rs])
        return 0
    lax.fori_loop(0, -(-nblocks // 16), _wr_blk, 0)


def _sc_edge(kvh, qh, weh, idx3, n_dst):
    nchunks = idx3.shape[0] - 1
    n_src = kvh.shape[0] // 2
    n_q = qh.shape[0] // 2
    nblocks = -(-n_dst // ROWB)
    nj = -(-nchunks // 16)
    assert n_q == n_dst

    mesh = plsc.VectorSubcoreMesh(core_axis_name="c", subcore_axis_name="s",
                                  num_cores=2, num_subcores=16)
    body = functools.partial(_sc_edge_body, nchunks, nblocks, n_src, n_q, nj)
    kfn = pl.kernel(
        body,
        out_type=jax.ShapeDtypeStruct((2, nblocks * ROWB, ACCW), jnp.float32),
        mesh=mesh,
        compiler_params=pltpu.CompilerParams(use_tc_tiling_on_sc=False,
                                             needs_layout_passes=False),
        scratch_types=[
            pltpu.VMEM_SHARED((nblocks * ROWB, ACCW), jnp.float32),
            pltpu.VMEM((2 * ED, C), jnp.float32),
            pltpu.VMEM((CH, 2 * C), jnp.float32),
            pltpu.VMEM((CH, 2 * C), jnp.float32),
            pltpu.VMEM((CH, C), jnp.float32),
            pltpu.VMEM((CH, C), jnp.float32),
            pltpu.VMEM((CH, ACCW), jnp.float32),
            pltpu.VMEM((CH, ACCW), jnp.float32),
            pltpu.VMEM((4, CH), jnp.int32),
            pltpu.VMEM((4, CH), jnp.int32),
            pltpu.VMEM((CH,), jnp.int32),
            pltpu.VMEM((CH,), jnp.int32),
            pltpu.VMEM((2, 3, CH), jnp.int32),
            pltpu.VMEM((2, 3, CH), jnp.int32),
            pltpu.SemaphoreType.DMA,
            pltpu.SemaphoreType.DMA,
            pltpu.SemaphoreType.DMA,
            pltpu.SemaphoreType.DMA,
            pltpu.SemaphoreType.DMA,
            pltpu.SemaphoreType.DMA,
            pltpu.SemaphoreType.DMA,
            pltpu.SemaphoreType.DMA,
        ],
    )
    return kfn(kvh, qh, weh, idx3)


def _gate1_body(acc_ref, xr_ref, ga_ref, gb_ref, lg_ref, lb_ref, o_ref):
    a0 = acc_ref[0]
    a1 = acc_ref[1]
    o0 = a0[:, :64] / (a0[:, 64:65] + 1e-16)
    o1 = a1[:, :64] / (a1[:, 64:65] + 1e-16)
    out = jnp.concatenate([o0, o1], axis=1)
    xr = xr_ref[...]
    beta = jax.nn.sigmoid(
        jnp.sum(xr * ga_ref[...] + out * gb_ref[...], axis=1, keepdims=True))
    h = beta * xr + (1.0 - beta) * out
    mu = jnp.mean(h, axis=1, keepdims=True)
    var = jnp.mean(jnp.square(h - mu), axis=1, keepdims=True)
    h = (h - mu) * jax.lax.rsqrt(var + 1e-5) * lg_ref[...] + lb_ref[...]
    o_ref[...] = jnp.maximum(h, 0.0)


def _gate1(acc, xr, ga, gb, lg, lb, block_rows=2000):
    n = xr.shape[0]
    vec = lambda v: v.reshape(1, -1)
    return pl.pallas_call(
        _gate1_body,
        grid=(n // block_rows,),
        in_specs=[
            pl.BlockSpec((2, block_rows, ACCW), lambda i: (0, i, 0)),
            pl.BlockSpec((block_rows, D), lambda i: (i, 0)),
            pl.BlockSpec((1, D), lambda i: (0, 0)),
            pl.BlockSpec((1, D), lambda i: (0, 0)),
            pl.BlockSpec((1, D), lambda i: (0, 0)),
            pl.BlockSpec((1, D), lambda i: (0, 0)),
        ],
        out_specs=pl.BlockSpec((block_rows, D), lambda i: (i, 0)),
        out_shape=jax.ShapeDtypeStruct((n, D), jnp.float32),
    )(acc, xr, vec(ga), vec(gb), vec(lg), vec(lb))


def _gate2_body(acc_ref, xr_ref, ga_ref, gb_ref, o_ref):
    a0 = acc_ref[0]
    a1 = acc_ref[1]
    o0 = a0[:, :64] / (a0[:, 64:65] + 1e-16)
    o1 = a1[:, :64] / (a1[:, 64:65] + 1e-16)
    out = 0.5 * (o0 + o1)
    xr = xr_ref[...]
    beta = jax.nn.sigmoid(
        jnp.sum(xr * ga_ref[...] + out * gb_ref[...], axis=1, keepdims=True))
    o = beta * xr + (1.0 - beta) * out
    m = jnp.max(o, axis=1, keepdims=True)
    lse = m + jnp.log(jnp.sum(jnp.exp(o - m), axis=1, keepdims=True))
    o_ref[...] = o - lse


def _gate2(acc, xr, ga, gb, block_rows=2000):
    n = xr.shape[0]
    vec = lambda v: v.reshape(1, -1)
    return pl.pallas_call(
        _gate2_body,
        grid=(n // block_rows,),
        in_specs=[
            pl.BlockSpec((2, block_rows, ACCW), lambda i: (0, i, 0)),
            pl.BlockSpec((block_rows, C), lambda i: (i, 0)),
            pl.BlockSpec((1, C), lambda i: (0, 0)),
            pl.BlockSpec((1, C), lambda i: (0, 0)),
        ],
        out_specs=pl.BlockSpec((block_rows, C), lambda i: (i, 0)),
        out_shape=jax.ShapeDtypeStruct((n, C), jnp.float32),
    )(acc, xr, vec(ga), vec(gb))


def _edge_blocks(src, dst, eid):
    e = src.shape[0]
    blk = jnp.stack([src.reshape(e // CH, CH), dst.reshape(e // CH, CH),
                     eid.reshape(e // CH, CH)], axis=1)
    return jnp.pad(blk, ((0, 1), (0, 0), (0, 0)))


def _weh(We):
    return We.reshape(ED, 2, C).transpose(1, 0, 2).reshape(2 * ED, C)


def kernel(x, src1, dst1, eid1, src2, dst2, eid2,
           Wq1, bq1, Wk1, bk1, Wv1, bv1, We1, Ws1, bs1, Wb1, ln_g, ln_b,
           Wq2, bq2, Wk2, bk2, Wv2, bv2, We2, Ws2, bs2, Wb2):
    kvh1 = _proj_kv(x, Wk1, bk1, Wv1, bv1).reshape(2 * N0, 2 * C)
    qh1, xr1 = _proj_qxr(x[:N1], Wq1, bq1, Ws1, bs1)
    qh1 = qh1.reshape(2 * N1, C)
    acc1 = _sc_edge(kvh1, qh1, _weh(We1),
                    _edge_blocks(src1, dst1, eid1[:, 0]), N1)
    ga1 = Wb1[:D, 0] + Wb1[2 * D:, 0]
    gb1 = Wb1[D:2 * D, 0] - Wb1[2 * D:, 0]
    h = _gate1(acc1, xr1, ga1, gb1, ln_g, ln_b)

    kvh2 = _proj_kv(h, Wk2, bk2, Wv2, bv2).reshape(2 * N1, 2 * C)
    qh2, xr2 = _proj_qxr(h[:N2], Wq2, bq2, Ws2, bs2)
    qh2 = qh2.reshape(2 * N2, C)
    acc2 = _sc_edge(kvh2, qh2, _weh(We2),
                    _edge_blocks(src2, dst2, eid2[:, 0]), N2)
    ga2 = Wb2[:C, 0] + Wb2[2 * C:, 0]
    gb2 = Wb2[C:2 * C, 0] - Wb2[2 * C:, 0]
    return _gate2(acc2, xr2, ga2, gb2)
